# baseline (speedup 1.0000x reference)
import functools

import jax
import jax.numpy as jnp
from jax import lax
from jax.experimental import pallas as pl
from jax.experimental.pallas import tpu as pltpu
from jax.experimental.pallas import tpu_sc as plsc

N_NODES = 10000
N_EDGES = 320000
NC, NS, LANES = 2, 16, 16
NW = NC * NS
EPW = N_EDGES // NW
EC = 80
NCH = EPW // EC
UNR = 16


def _worker_ids():
    c = lax.axis_index("c")
    s = lax.axis_index("s")
    return c, s, s * NC + c


def _deg_body(dst_hbm, ew_hbm, out_hbm, dstb, ewb, zb, dsem, acc):
    c, s, w = _worker_ids()

    def zrow(i, _):
        zb[pl.ds(i * 16, 16)] = jnp.zeros((16,), jnp.float32)
        return 0
    lax.fori_loop(0, 40, zrow, 0)

    @pl.when(s < NS - 1)
    def _():
        pltpu.sync_copy(zb.at[pl.ds(0, 624)],
                        acc.at[pl.ds(pl.multiple_of(s * 624, 8), 624)])

    @pl.when(s == NS - 1)
    def _():
        pltpu.sync_copy(zb, acc.at[pl.ds((NS - 1) * 624, 640)])

    plsc.subcore_barrier()

    def chunk(i, _):
        r0 = w * (EPW // 80) + i * 5
        pltpu.sync_copy(dst_hbm.at[pl.ds(r0, 5)], dstb)
        pltpu.sync_copy(ew_hbm.at[pl.ds(r0, 5)], ewb)
        for j in range(5):
            pltpu.async_copy(ewb.at[j], acc.at[dstb.at[j]], dsem, add=True)
        for j in range(5):
            pltpu.make_async_copy(ewb.at[0], acc.at[dstb.at[0]], dsem).wait()
        return 0
    lax.fori_loop(0, 25, chunk, 0)

    plsc.subcore_barrier()

    @pl.when(s < NS - 1)
    def _():
        o = pl.multiple_of(s * 624, 8)
        pltpu.sync_copy(acc.at[pl.ds(o, 624)], zb.at[pl.ds(0, 624)])
        pltpu.sync_copy(zb.at[pl.ds(0, 624)],
                        out_hbm.at[pl.ds(c * N_NODES + o, 624)])

    @pl.when(s == NS - 1)
    def _():
        o = pl.multiple_of(c * N_NODES + (NS - 1) * 624, 8)
        pltpu.sync_copy(acc.at[pl.ds((NS - 1) * 624, 640)], zb)
        pltpu.sync_copy(zb, out_hbm.at[pl.ds(o, 640)])


_deg_call = pl.kernel(
    _deg_body,
    out_type=jax.ShapeDtypeStruct((NC * N_NODES,), jnp.float32),
    mesh=plsc.VectorSubcoreMesh(core_axis_name="c", subcore_axis_name="s"),
    compiler_params=pltpu.CompilerParams(use_tc_tiling_on_sc=False,
                                         needs_layout_passes=False),
    scratch_types=[
        pltpu.VMEM((5, 80), jnp.int32),
        pltpu.VMEM((5, 80), jnp.float32),
        pltpu.VMEM((640,), jnp.float32),
        pltpu.SemaphoreType.DMA,
        pltpu.VMEM_SHARED((N_NODES,), jnp.float32),
    ],
)


NC2 = 400
NCH2 = EPW // NC2


def _norm_body(src_hbm, dst_hbm, ew_hbm, dinv_hbm, out_hbm, dv, srcb, dstb, ewb, nbi):
    _, _, w = _worker_ids()
    pltpu.sync_copy(dinv_hbm, dv)

    def chunk(i, _):
        off = pl.multiple_of(w * EPW + i * NC2, 8)
        pltpu.sync_copy(src_hbm.at[pl.ds(off, NC2)], srcb)
        pltpu.sync_copy(dst_hbm.at[pl.ds(off, NC2)], dstb)
        pltpu.sync_copy(ew_hbm.at[pl.ds(off, NC2)], ewb)

        def inner(j, _):
            sl = pl.ds(j * 16, 16)
            nv = (plsc.load_gather(dv, [srcb[sl]]) * ewb[sl]
                  * plsc.load_gather(dv, [dstb[sl]]))
            nbi[sl] = plsc.bitcast(nv, jnp.int32)
            return 0
        lax.fori_loop(0, NC2 // 16, inner, 0)
        pltpu.sync_copy(srcb, out_hbm.at[0, pl.ds(off, NC2)])
        pltpu.sync_copy(dstb, out_hbm.at[1, pl.ds(off, NC2)])
        pltpu.sync_copy(nbi, out_hbm.at[2, pl.ds(off, NC2)])
        return 0
    lax.fori_loop(0, NCH2, chunk, 0)


_norm_call = pl.kernel(
    _norm_body,
    out_type=jax.ShapeDtypeStruct((3, N_EDGES), jnp.int32),
    mesh=plsc.VectorSubcoreMesh(core_axis_name="c", subcore_axis_name="s"),
    compiler_params=pltpu.CompilerParams(use_tc_tiling_on_sc=False,
                                         needs_layout_passes=False),
    scratch_types=[
        pltpu.VMEM((N_NODES,), jnp.float32),
        pltpu.VMEM((NC2,), jnp.int32),
        pltpu.VMEM((NC2,), jnp.int32),
        pltpu.VMEM((NC2,), jnp.float32),
        pltpu.VMEM((NC2,), jnp.int32),
    ],
)


def _spmm_body(F, x_hbm, pk_hbm, out_hbm,
               pk0, pk1, pk2, pk3, rows0, rows1, rows2, rows3,
               isem0, isem1, isem2, isem3, gsem0, gsem1, gsem2, gsem3,
               ssem0, ssem1, ssem2, ssem3, acc):
    c, s, w = _worker_ids()
    bufs = ((pk0, rows0, isem0, gsem0, ssem0),
            (pk1, rows1, isem1, gsem1, ssem1),
            (pk2, rows2, isem2, gsem2, ssem2),
            (pk3, rows3, isem3, gsem3, ssem3))

    def zrow(i, _):
        for j in range(F // 16):
            rows0[i, pl.ds(j * 16, 16)] = jnp.zeros((16,), jnp.float32)
        return 0
    lax.fori_loop(0, EC, zrow, 0)
    for k in range(7):
        pltpu.sync_copy(rows0, acc.at[pl.ds(s * 624 + k * 80, 80)])
    pltpu.sync_copy(rows0.at[pl.ds(0, 64)], acc.at[pl.ds(s * 624 + 560, 64)])

    @pl.when(s == NS - 1)
    def _():
        pltpu.sync_copy(rows0.at[pl.ds(0, 16)], acc.at[pl.ds(N_NODES - 16, 16)])

    plsc.subcore_barrier()

    def _idx_fetch(i, b):
        pk, rows, isem, gsem, ssem = bufs[b]
        off = pl.multiple_of(w * EPW + i * EC, 8)
        pltpu.async_copy(pk_hbm.at[:, pl.ds(off, EC)], pk, isem)

    def _idx_wait(b):
        pk, rows, isem, gsem, ssem = bufs[b]
        pltpu.make_async_copy(pk_hbm.at[:, pl.ds(0, EC)], pk, isem).wait()

    def _gather(b):
        pk, rows, isem, gsem, ssem = bufs[b]
        pltpu.async_copy(x_hbm.at[pk.at[0]], rows, gsem)

    def _gather_wait(b):
        pk, rows, isem, gsem, ssem = bufs[b]
        pltpu.make_async_copy(x_hbm.at[pk.at[0]], rows, gsem).wait()

    def _scale(b, nrows):
        pk, rows, isem, gsem, ssem = bufs[b]

        def srow(g, _):
            for u in range(UNR):
                r = g * UNR + u
                nbc = plsc.bitcast(
                    plsc.load_gather(pk.at[2], [jnp.full((LANES,), r, jnp.int32)]),
                    jnp.float32)
                for j in range(F // 16):
                    rows[r, pl.ds(j * 16, 16)] = rows[r, pl.ds(j * 16, 16)] * nbc
            return 0
        lax.fori_loop(0, nrows // UNR, srow, 0)

    def _scatter(b):
        pk, rows, isem, gsem, ssem = bufs[b]
        pltpu.async_copy(rows, acc.at[pk.at[1]], ssem, add=True)

    def _wait_scatter(b):
        pk, rows, isem, gsem, ssem = bufs[b]
        pltpu.make_async_copy(rows, acc.at[pk.at[1]], ssem).wait()

    _idx_fetch(0, 0)
    _idx_wait(0)
    _gather(0)
    _idx_fetch(1, 1)
    _idx_wait(1)
    _gather(1)
    _idx_fetch(2, 2)

    def quad(k, _):
        for b in range(4):
            i = 4 * k + b
            _gather_wait(b)
            nx2 = (b + 2) % 4
            pre = (b + 3) % 4

            @pl.when(i + 2 < NCH)
            def _():
                _idx_wait(nx2)
                _gather(nx2)

            @pl.when(i >= 1)
            def _():
                _wait_scatter(pre)

            @pl.when(i + 3 < NCH)
            def _():
                _idx_fetch(i + 3, pre)
            _scale(b, EC)
            _scatter(b)
        return 0
    lax.fori_loop(0, NCH // 4, quad, 0)

    _gather_wait(0)
    _wait_scatter(3)
    _scale(0, EC)
    _scatter(0)
    _wait_scatter(0)

    plsc.subcore_barrier()

    for k in range(7):
        o = pl.multiple_of(s * 624 + k * 80, 8)
        stg, wsem = bufs[k % 4][1], bufs[k % 4][3]
        if k >= 4:
            pltpu.make_async_copy(stg, out_hbm.at[c, pl.ds(0, 80)], wsem).wait()
        pltpu.sync_copy(acc.at[pl.ds(o, 80)], stg)
        pltpu.async_copy(stg, out_hbm.at[c, pl.ds(o, 80)], wsem)
    for k in range(4):
        stg, wsem = bufs[k][1], bufs[k][3]
        pltpu.make_async_copy(stg, out_hbm.at[c, pl.ds(0, 80)], wsem).wait()
    stg, wsem = bufs[3][1], bufs[3][3]
    pltpu.sync_copy(acc.at[pl.ds(s * 624 + 560, 64)], stg.at[pl.ds(0, 64)])
    pltpu.sync_copy(stg.at[pl.ds(0, 64)],
                    out_hbm.at[c, pl.ds(s * 624 + 560, 64)])

    @pl.when(s == NS - 1)
    def _():
        pltpu.sync_copy(acc.at[pl.ds(N_NODES - 16, 16)], rows0.at[pl.ds(0, 16)])
        pltpu.sync_copy(rows0.at[pl.ds(0, 16)],
                        out_hbm.at[c, pl.ds(N_NODES - 16, 16)])


@functools.cache
def _make_spmm(F):
    return pl.kernel(
        functools.partial(_spmm_body, F),
        out_type=jax.ShapeDtypeStruct((NC, N_NODES, F), jnp.float32),
        mesh=plsc.VectorSubcoreMesh(core_axis_name="c", subcore_axis_name="s"),
        compiler_params=pltpu.CompilerParams(use_tc_tiling_on_sc=False,
                                             needs_layout_passes=False),
        scratch_types=(
            [pltpu.VMEM((3, EC), jnp.int32)] * 4
            + [pltpu.VMEM((EC, F), jnp.float32)] * 4
            + [pltpu.SemaphoreType.DMA] * 12
            + [pltpu.VMEM_SHARED((N_NODES, F), jnp.float32)]
        ),
    )


BS = 1000


def _dense_body(relu, y_ref, x_ref, d2_ref, w_ref, b_ref, o_ref):
    h = y_ref[0] + y_ref[1] + d2_ref[...] * x_ref[...]
    o = lax.dot_general(h, w_ref[...], (((1,), (0,)), ((), ())),
                        precision=lax.Precision.HIGHEST,
                        preferred_element_type=jnp.float32) + b_ref[...]
    o_ref[...] = jnp.maximum(o, 0.0) if relu else o


def _dense(y, x, d2, W, b, relu):
    n, fin = x.shape
    fout = W.shape[1]
    return pl.pallas_call(
        functools.partial(_dense_body, relu),
        grid=(n // BS,),
        in_specs=[
            pl.BlockSpec((NC, BS, fin), lambda i: (0, i, 0)),
            pl.BlockSpec((BS, fin), lambda i: (i, 0)),
            pl.BlockSpec((BS, 1), lambda i: (i, 0)),
            pl.BlockSpec((fin, fout), lambda i: (0, 0)),
            pl.BlockSpec((1, fout), lambda i: (0, 0)),
        ],
        out_specs=pl.BlockSpec((BS, fout), lambda i: (i, 0)),
        out_shape=jax.ShapeDtypeStruct((n, fout), jnp.float32),
    )(y, x, d2, W, b.reshape(1, -1))


def _stageb_body(y_ref, h_ref, d2_ref, wmu_ref, bmu_ref, wlv_ref, blv_ref,
                 eps_ref, mu_ref, lv_ref, z_ref):
    H = y_ref[0] + y_ref[1] + d2_ref[...] * h_ref[...]
    dn = (((1,), (0,)), ((), ()))
    mu = lax.dot_general(H, wmu_ref[...], dn, precision=lax.Precision.HIGHEST,
                         preferred_element_type=jnp.float32) + bmu_ref[...]
    lv = lax.dot_general(H, wlv_ref[...], dn, precision=lax.Precision.HIGHEST,
                         preferred_element_type=jnp.float32) + blv_ref[...]
    mu_ref[...] = mu
    lv_ref[...] = lv
    z_ref[...] = mu + eps_ref[...] * jnp.exp(0.5 * lv)


def _stageb(y, hidden, d2, mu_W, mu_b, lv_W, lv_b, eps):
    n, fin = hidden.shape
    fout = mu_W.shape[1]
    return pl.pallas_call(
        _stageb_body,
        grid=(n // BS,),
        in_specs=[
            pl.BlockSpec((NC, BS, fin), lambda i: (0, i, 0)),
            pl.BlockSpec((BS, fin), lambda i: (i, 0)),
            pl.BlockSpec((BS, 1), lambda i: (i, 0)),
            pl.BlockSpec((fin, fout), lambda i: (0, 0)),
            pl.BlockSpec((1, fout), lambda i: (0, 0)),
            pl.BlockSpec((fin, fout), lambda i: (0, 0)),
            pl.BlockSpec((1, fout), lambda i: (0, 0)),
            pl.BlockSpec((BS, fout), lambda i: (i, 0)),
        ],
        out_specs=[
            pl.BlockSpec((BS, fout), lambda i: (i, 0)),
            pl.BlockSpec((BS, fout), lambda i: (i, 0)),
            pl.BlockSpec((BS, fout), lambda i: (i, 0)),
        ],
        out_shape=[
            jax.ShapeDtypeStruct((n, fout), jnp.float32),
            jax.ShapeDtypeStruct((n, fout), jnp.float32),
            jax.ShapeDtypeStruct((n, fout), jnp.float32),
        ],
    )(y, hidden, d2, mu_W, mu_b.reshape(1, -1), lv_W, lv_b.reshape(1, -1), eps)


def kernel(x, edge_index, edge_attr, enc1_W, enc1_b, mu_W, mu_b, lv_W, lv_b,
           dec1_W, dec1_b, dec2_W, dec2_b):
    ei = edge_index.astype(jnp.int32)
    src = ei[0]
    dst = ei[1]
    ew = edge_attr.astype(jnp.float32)

    degp = _deg_call(dst.reshape(-1, 80),
                     ew.reshape(-1, 80)).reshape(NC, N_NODES)
    deg = degp[0] + degp[1] + 1.0
    dinv = jnp.where(deg > 0, lax.rsqrt(deg), 0.0)
    d2 = (dinv * dinv)[:, None]
    pk = _norm_call(src, dst, ew, dinv)

    spmm128 = _make_spmm(128)
    spmm64 = _make_spmm(64)

    y1 = spmm128(x, pk)
    hidden = _dense(y1, x, d2, enc1_W, enc1_b, True)

    y2 = spmm128(hidden, pk)
    eps = jax.random.normal(jax.random.key(42), (N_NODES, mu_W.shape[1]),
                            jnp.float32)
    mu, logvar, z = _stageb(y2, hidden, d2, mu_W, mu_b, lv_W, lv_b, eps)

    y3 = spmm64(z, pk)
    dh = _dense(y3, z, d2, dec1_W, dec1_b, True)

    y4 = spmm128(dh, pk)
    reconstructed_x = _dense(y4, dh, d2, dec2_W, dec2_b, False)
    return (reconstructed_x, mu, logvar)


# speedup vs baseline: 1.1464x; 1.1464x over previous
import functools

import jax
import jax.numpy as jnp
from jax import lax
from jax.experimental import pallas as pl
from jax.experimental.pallas import tpu as pltpu
from jax.experimental.pallas import tpu_sc as plsc

N_NODES = 10000
N_EDGES = 320000
NC, NS, LANES = 2, 16, 16
NW = NC * NS
EPW = N_EDGES // NW
EC = 80
NCH = EPW // EC
UNR = 8


def _worker_ids():
    c = lax.axis_index("c")
    s = lax.axis_index("s")
    return c, s, s * NC + c


def _deg_body(dst_hbm, ew_hbm, out_hbm, dstb, ewb, zb, dsem, acc):
    c, s, w = _worker_ids()

    def zrow(i, _):
        zb[pl.ds(i * 16, 16)] = jnp.zeros((16,), jnp.float32)
        return 0
    lax.fori_loop(0, 40, zrow, 0)

    @pl.when(s < NS - 1)
    def _():
        pltpu.sync_copy(zb.at[pl.ds(0, 624)],
                        acc.at[pl.ds(pl.multiple_of(s * 624, 8), 624)])

    @pl.when(s == NS - 1)
    def _():
        pltpu.sync_copy(zb, acc.at[pl.ds((NS - 1) * 624, 640)])

    plsc.subcore_barrier()

    def chunk(i, _):
        r0 = w * (EPW // 80) + i * 5
        pltpu.sync_copy(dst_hbm.at[pl.ds(r0, 5)], dstb)
        pltpu.sync_copy(ew_hbm.at[pl.ds(r0, 5)], ewb)
        for j in range(5):
            pltpu.async_copy(ewb.at[j], acc.at[dstb.at[j]], dsem, add=True)
        for j in range(5):
            pltpu.make_async_copy(ewb.at[0], acc.at[dstb.at[0]], dsem).wait()
        return 0
    lax.fori_loop(0, 25, chunk, 0)

    plsc.subcore_barrier()

    @pl.when(s < NS - 1)
    def _():
        o = pl.multiple_of(s * 624, 8)
        pltpu.sync_copy(acc.at[pl.ds(o, 624)], zb.at[pl.ds(0, 624)])
        pltpu.sync_copy(zb.at[pl.ds(0, 624)],
                        out_hbm.at[pl.ds(c * N_NODES + o, 624)])

    @pl.when(s == NS - 1)
    def _():
        o = pl.multiple_of(c * N_NODES + (NS - 1) * 624, 8)
        pltpu.sync_copy(acc.at[pl.ds((NS - 1) * 624, 640)], zb)
        pltpu.sync_copy(zb, out_hbm.at[pl.ds(o, 640)])


_deg_call = pl.kernel(
    _deg_body,
    out_type=jax.ShapeDtypeStruct((NC * N_NODES,), jnp.float32),
    mesh=plsc.VectorSubcoreMesh(core_axis_name="c", subcore_axis_name="s"),
    compiler_params=pltpu.CompilerParams(use_tc_tiling_on_sc=False,
                                         needs_layout_passes=False),
    scratch_types=[
        pltpu.VMEM((5, 80), jnp.int32),
        pltpu.VMEM((5, 80), jnp.float32),
        pltpu.VMEM((640,), jnp.float32),
        pltpu.SemaphoreType.DMA,
        pltpu.VMEM_SHARED((N_NODES,), jnp.float32),
    ],
)


NC2 = 400
NCH2 = EPW // NC2


def _norm_body(src_hbm, dst_hbm, ew_hbm, dinv_hbm, out_hbm, dv, srcb, dstb, ewb, nbi):
    _, _, w = _worker_ids()
    pltpu.sync_copy(dinv_hbm, dv)

    def chunk(i, _):
        off = pl.multiple_of(w * EPW + i * NC2, 8)
        pltpu.sync_copy(src_hbm.at[pl.ds(off, NC2)], srcb)
        pltpu.sync_copy(dst_hbm.at[pl.ds(off, NC2)], dstb)
        pltpu.sync_copy(ew_hbm.at[pl.ds(off, NC2)], ewb)

        def inner(j, _):
            sl = pl.ds(j * 16, 16)
            nv = (plsc.load_gather(dv, [srcb[sl]]) * ewb[sl]
                  * plsc.load_gather(dv, [dstb[sl]]))
            nbi[sl] = plsc.bitcast(nv, jnp.int32)
            return 0
        lax.fori_loop(0, NC2 // 16, inner, 0)
        pltpu.sync_copy(srcb, out_hbm.at[0, pl.ds(off, NC2)])
        pltpu.sync_copy(dstb, out_hbm.at[1, pl.ds(off, NC2)])
        pltpu.sync_copy(nbi, out_hbm.at[2, pl.ds(off, NC2)])
        return 0
    lax.fori_loop(0, NCH2, chunk, 0)


_norm_call = pl.kernel(
    _norm_body,
    out_type=jax.ShapeDtypeStruct((3, N_EDGES), jnp.int32),
    mesh=plsc.VectorSubcoreMesh(core_axis_name="c", subcore_axis_name="s"),
    compiler_params=pltpu.CompilerParams(use_tc_tiling_on_sc=False,
                                         needs_layout_passes=False),
    scratch_types=[
        pltpu.VMEM((N_NODES,), jnp.float32),
        pltpu.VMEM((NC2,), jnp.int32),
        pltpu.VMEM((NC2,), jnp.int32),
        pltpu.VMEM((NC2,), jnp.float32),
        pltpu.VMEM((NC2,), jnp.int32),
    ],
)


def _spmm_body(F, x_hbm, pk_hbm, out_hbm,
               pk0, pk1, pk2, pk3, rows0, rows1, rows2, rows3,
               isem0, isem1, isem2, isem3, gsem0, gsem1, gsem2, gsem3,
               ssem0, ssem1, ssem2, ssem3, acc):
    c, s, w = _worker_ids()
    bufs = ((pk0, rows0, isem0, gsem0, ssem0),
            (pk1, rows1, isem1, gsem1, ssem1),
            (pk2, rows2, isem2, gsem2, ssem2),
            (pk3, rows3, isem3, gsem3, ssem3))

    def zrow(i, _):
        for j in range(F // 16):
            rows0[i, pl.ds(j * 16, 16)] = jnp.zeros((16,), jnp.float32)
        return 0
    lax.fori_loop(0, EC, zrow, 0)
    for k in range(7):
        pltpu.sync_copy(rows0, acc.at[pl.ds(s * 624 + k * 80, 80)])
    pltpu.sync_copy(rows0.at[pl.ds(0, 64)], acc.at[pl.ds(s * 624 + 560, 64)])

    @pl.when(s == NS - 1)
    def _():
        pltpu.sync_copy(rows0.at[pl.ds(0, 16)], acc.at[pl.ds(N_NODES - 16, 16)])

    plsc.subcore_barrier()

    def _idx_fetch(i, b):
        pk, rows, isem, gsem, ssem = bufs[b]
        off = pl.multiple_of(w * EPW + i * EC, 8)
        pltpu.async_copy(pk_hbm.at[:, pl.ds(off, EC)], pk, isem)

    def _idx_wait(b):
        pk, rows, isem, gsem, ssem = bufs[b]
        pltpu.make_async_copy(pk_hbm.at[:, pl.ds(0, EC)], pk, isem).wait()

    def _gather(b):
        pk, rows, isem, gsem, ssem = bufs[b]
        pltpu.async_copy(x_hbm.at[pk.at[0]], rows, gsem)

    def _gather_wait(b):
        pk, rows, isem, gsem, ssem = bufs[b]
        pltpu.make_async_copy(x_hbm.at[pk.at[0]], rows, gsem).wait()

    def _scale(b, nrows):
        pk, rows, isem, gsem, ssem = bufs[b]

        def srow(g, _):
            for u in range(UNR):
                r = g * UNR + u
                nbc = plsc.bitcast(
                    plsc.load_gather(pk.at[2], [jnp.full((LANES,), r, jnp.int32)]),
                    jnp.float32)
                for j in range(F // 16):
                    rows[r, pl.ds(j * 16, 16)] = rows[r, pl.ds(j * 16, 16)] * nbc
            return 0
        lax.fori_loop(0, nrows // UNR, srow, 0)

    def _scatter(b):
        pk, rows, isem, gsem, ssem = bufs[b]
        pltpu.async_copy(rows, acc.at[pk.at[1]], ssem, add=True)

    def _wait_scatter(b):
        pk, rows, isem, gsem, ssem = bufs[b]
        pltpu.make_async_copy(rows, acc.at[pk.at[1]], ssem).wait()

    _idx_fetch(0, 0)
    _idx_wait(0)
    _gather(0)
    _idx_fetch(1, 1)
    _idx_wait(1)
    _gather(1)
    _idx_fetch(2, 2)

    def quad(k, _):
        for b in range(4):
            i = 4 * k + b
            _gather_wait(b)
            nx2 = (b + 2) % 4
            pre = (b + 3) % 4

            @pl.when(i + 2 < NCH)
            def _():
                _idx_wait(nx2)
                _gather(nx2)

            @pl.when(i >= 1)
            def _():
                _wait_scatter(pre)

            @pl.when(i + 3 < NCH)
            def _():
                _idx_fetch(i + 3, pre)
            _scale(b, EC)
            _scatter(b)
        return 0
    lax.fori_loop(0, NCH // 4, quad, 0)

    _gather_wait(0)
    _wait_scatter(3)
    _scale(0, EC)
    _scatter(0)
    _wait_scatter(0)

    plsc.subcore_barrier()

    for k in range(7):
        o = pl.multiple_of(s * 624 + k * 80, 8)
        stg, wsem = bufs[k % 4][1], bufs[k % 4][3]
        if k >= 4:
            pltpu.make_async_copy(stg, out_hbm.at[c, pl.ds(0, 80)], wsem).wait()
        pltpu.sync_copy(acc.at[pl.ds(o, 80)], stg)
        pltpu.async_copy(stg, out_hbm.at[c, pl.ds(o, 80)], wsem)
    for k in range(4):
        stg, wsem = bufs[k][1], bufs[k][3]
        pltpu.make_async_copy(stg, out_hbm.at[c, pl.ds(0, 80)], wsem).wait()
    stg, wsem = bufs[3][1], bufs[3][3]
    pltpu.sync_copy(acc.at[pl.ds(s * 624 + 560, 64)], stg.at[pl.ds(0, 64)])
    pltpu.sync_copy(stg.at[pl.ds(0, 64)],
                    out_hbm.at[c, pl.ds(s * 624 + 560, 64)])

    @pl.when(s == NS - 1)
    def _():
        pltpu.sync_copy(acc.at[pl.ds(N_NODES - 16, 16)], rows0.at[pl.ds(0, 16)])
        pltpu.sync_copy(rows0.at[pl.ds(0, 16)],
                        out_hbm.at[c, pl.ds(N_NODES - 16, 16)])


@functools.cache
def _make_spmm(F):
    return pl.kernel(
        functools.partial(_spmm_body, F),
        out_type=jax.ShapeDtypeStruct((NC, N_NODES, F), jnp.float32),
        mesh=plsc.VectorSubcoreMesh(core_axis_name="c", subcore_axis_name="s"),
        compiler_params=pltpu.CompilerParams(use_tc_tiling_on_sc=False,
                                             needs_layout_passes=False),
        scratch_types=(
            [pltpu.VMEM((3, EC), jnp.int32)] * 4
            + [pltpu.VMEM((EC, F), jnp.float32)] * 4
            + [pltpu.SemaphoreType.DMA] * 12
            + [pltpu.VMEM_SHARED((N_NODES, F), jnp.float32)]
        ),
    )


BS = 1000


def _dense_body(relu, y_ref, x_ref, d2_ref, w_ref, b_ref, o_ref):
    h = y_ref[0] + y_ref[1] + d2_ref[...] * x_ref[...]
    o = lax.dot_general(h, w_ref[...], (((1,), (0,)), ((), ())),
                        precision=lax.Precision.HIGHEST,
                        preferred_element_type=jnp.float32) + b_ref[...]
    o_ref[...] = jnp.maximum(o, 0.0) if relu else o


def _dense(y, x, d2, W, b, relu):
    n, fin = x.shape
    fout = W.shape[1]
    return pl.pallas_call(
        functools.partial(_dense_body, relu),
        grid=(n // BS,),
        in_specs=[
            pl.BlockSpec((NC, BS, fin), lambda i: (0, i, 0)),
            pl.BlockSpec((BS, fin), lambda i: (i, 0)),
            pl.BlockSpec((BS, 1), lambda i: (i, 0)),
            pl.BlockSpec((fin, fout), lambda i: (0, 0)),
            pl.BlockSpec((1, fout), lambda i: (0, 0)),
        ],
        out_specs=pl.BlockSpec((BS, fout), lambda i: (i, 0)),
        out_shape=jax.ShapeDtypeStruct((n, fout), jnp.float32),
    )(y, x, d2, W, b.reshape(1, -1))


def _stageb_body(y_ref, h_ref, d2_ref, wmu_ref, bmu_ref, wlv_ref, blv_ref,
                 eps_ref, mu_ref, lv_ref, z_ref):
    H = y_ref[0] + y_ref[1] + d2_ref[...] * h_ref[...]
    dn = (((1,), (0,)), ((), ()))
    mu = lax.dot_general(H, wmu_ref[...], dn, precision=lax.Precision.HIGHEST,
                         preferred_element_type=jnp.float32) + bmu_ref[...]
    lv = lax.dot_general(H, wlv_ref[...], dn, precision=lax.Precision.HIGHEST,
                         preferred_element_type=jnp.float32) + blv_ref[...]
    mu_ref[...] = mu
    lv_ref[...] = lv
    z_ref[...] = mu + eps_ref[...] * jnp.exp(0.5 * lv)


def _stageb(y, hidden, d2, mu_W, mu_b, lv_W, lv_b, eps):
    n, fin = hidden.shape
    fout = mu_W.shape[1]
    return pl.pallas_call(
        _stageb_body,
        grid=(n // BS,),
        in_specs=[
            pl.BlockSpec((NC, BS, fin), lambda i: (0, i, 0)),
            pl.BlockSpec((BS, fin), lambda i: (i, 0)),
            pl.BlockSpec((BS, 1), lambda i: (i, 0)),
            pl.BlockSpec((fin, fout), lambda i: (0, 0)),
            pl.BlockSpec((1, fout), lambda i: (0, 0)),
            pl.BlockSpec((fin, fout), lambda i: (0, 0)),
            pl.BlockSpec((1, fout), lambda i: (0, 0)),
            pl.BlockSpec((BS, fout), lambda i: (i, 0)),
        ],
        out_specs=[
            pl.BlockSpec((BS, fout), lambda i: (i, 0)),
            pl.BlockSpec((BS, fout), lambda i: (i, 0)),
            pl.BlockSpec((BS, fout), lambda i: (i, 0)),
        ],
        out_shape=[
            jax.ShapeDtypeStruct((n, fout), jnp.float32),
            jax.ShapeDtypeStruct((n, fout), jnp.float32),
            jax.ShapeDtypeStruct((n, fout), jnp.float32),
        ],
    )(y, hidden, d2, mu_W, mu_b.reshape(1, -1), lv_W, lv_b.reshape(1, -1), eps)


def kernel(x, edge_index, edge_attr, enc1_W, enc1_b, mu_W, mu_b, lv_W, lv_b,
           dec1_W, dec1_b, dec2_W, dec2_b):
    ei = edge_index.astype(jnp.int32)
    src = ei[0]
    dst = ei[1]
    ew = edge_attr.astype(jnp.float32)

    degp = _deg_call(dst.reshape(-1, 80),
                     ew.reshape(-1, 80)).reshape(NC, N_NODES)
    deg = degp[0] + degp[1] + 1.0
    dinv = jnp.where(deg > 0, lax.rsqrt(deg), 0.0)
    d2 = (dinv * dinv)[:, None]
    pk = _norm_call(src, dst, ew, dinv)

    spmm128 = _make_spmm(128)
    spmm64 = _make_spmm(64)

    y1 = spmm128(x, pk)
    hidden = _dense(y1, x, d2, enc1_W, enc1_b, True)

    y2 = spmm128(hidden, pk)
    eps = jax.random.normal(jax.random.key(42), (N_NODES, mu_W.shape[1]),
                            jnp.float32)
    mu, logvar, z = _stageb(y2, hidden, d2, mu_W, mu_b, lv_W, lv_b, eps)

    y3 = spmm64(z, pk)
    dh = _dense(y3, z, d2, dec1_W, dec1_b, True)

    y4 = spmm128(dh, pk)
    reconstructed_x = _dense(y4, dh, d2, dec2_W, dec2_b, False)
    return (reconstructed_x, mu, logvar)
